# Initial kernel scaffold; baseline (speedup 1.0000x reference)
import jax
import jax.numpy as jnp
from jax.experimental import pallas as pl


def kernel(z_e, codebook):
    raise NotImplementedError("write your pallas kernel here")


# speedup vs baseline: 3.8981x; 3.8981x over previous
import jax
import jax.numpy as jnp
from jax.experimental import pallas as pl

NUM_CODES = 1024
CODE_DIM = 64

_HI = jax.lax.Precision.HIGHEST


def _body(z_ref, c_ref, ct_ref, zq_ref, idx_ref):
    z = z_ref[...]
    c = c_ref[...]
    ct = ct_ref[...]
    K = NUM_CODES
    cn = jnp.sum(ct * ct, axis=0)[None, :]
    zc = jax.lax.dot_general(z, ct, (((1,), (0,)), ((), ())),
                             precision=_HI, preferred_element_type=jnp.float32)
    S = cn - 2.0 * zc
    T = z.shape[0]
    iota = jax.lax.broadcasted_iota(jnp.int32, (T, K), 1)
    m1 = jnp.min(S, axis=1, keepdims=True)
    i1 = jnp.min(jnp.where(S == m1, iota, K), axis=1, keepdims=True)
    S2 = jnp.where(iota == i1, jnp.inf, S)
    m2 = jnp.min(S2, axis=1, keepdims=True)
    i2 = jnp.min(jnp.where(S2 == m2, iota, K), axis=1, keepdims=True)
    oh1 = (iota == i1).astype(jnp.float32)
    oh2 = (iota == i2).astype(jnp.float32)
    r1 = jax.lax.dot_general(oh1, c, (((1,), (0,)), ((), ())),
                             precision=_HI, preferred_element_type=jnp.float32)
    r2 = jax.lax.dot_general(oh2, c, (((1,), (0,)), ((), ())),
                             precision=_HI, preferred_element_type=jnp.float32)
    d1 = jnp.sqrt(jnp.sum((z - r1) ** 2, axis=1, keepdims=True))
    d2 = jnp.sqrt(jnp.sum((z - r2) ** 2, axis=1, keepdims=True))
    take2 = (d2 < d1) | ((d2 == d1) & (i2 < i1))
    idx_ref[...] = jnp.where(take2, i2, i1)
    zq_ref[...] = jnp.where(take2, r2, r1)


TM = 128


def kernel(z_e, codebook):
    B, S_len, D = z_e.shape
    T = B * S_len
    z2 = z_e.reshape(T, D)
    zq, idx = pl.pallas_call(
        _body,
        grid=(T // TM,),
        in_specs=[
            pl.BlockSpec((TM, D), lambda i: (i, 0)),
            pl.BlockSpec((NUM_CODES, D), lambda i: (0, 0)),
            pl.BlockSpec((D, NUM_CODES), lambda i: (0, 0)),
        ],
        out_specs=(
            pl.BlockSpec((TM, D), lambda i: (i, 0)),
            pl.BlockSpec((TM, 1), lambda i: (i, 0)),
        ),
        out_shape=(
            jax.ShapeDtypeStruct((T, D), jnp.float32),
            jax.ShapeDtypeStruct((T, 1), jnp.int32),
        ),
    )(z2, codebook, codebook.T)
    return (zq.reshape(B, S_len, D), idx.reshape(B, S_len))
